# Initial kernel scaffold; baseline (speedup 1.0000x reference)
import jax
import jax.numpy as jnp
from jax.experimental import pallas as pl


def kernel(x, upflow):
    raise NotImplementedError("write your pallas kernel here")


# speedup vs baseline: 3.8694x; 3.8694x over previous
import jax
import jax.numpy as jnp
from jax.experimental import pallas as pl
from jax.experimental.pallas import tpu as pltpu

_ROWS_PER_STEP = 8
_ROWS_PER_WRITE = 8
_DPACK = 4


def _warp_kernel(x_ref, flow_ref, outx_ref, outm_ref, accx_ref, accm_ref):
    _, C, RB, W = x_ref.shape
    H = accx_ref.shape[0]
    RW = outx_ref.shape[1]
    n_acc = H // RB
    i = pl.program_id(1)

    @pl.when(i == 0)
    def _init():
        accx_ref[...] = jnp.zeros_like(accx_ref)
        accm_ref[...] = jnp.zeros_like(accm_ref)

    @pl.when(i >= n_acc)
    def _writeout():
        wb = i - n_acc
        outx_ref[0] = accx_ref[pl.ds(wb * RW, RW)]
        outm_ref[0] = accm_ref[pl.ds(wb * RW, RW)]

    @pl.when(i < n_acc)
    def _accumulate():
        px = jax.lax.broadcasted_iota(jnp.int32, (1, W), 1).astype(jnp.float32)
        csub = jax.lax.broadcasted_iota(jnp.int32, (W, W), 0).astype(jnp.float32)
        ksub = jax.lax.broadcasted_iota(jnp.int32, (8, W), 0)

        def process_row(xrow, tx, ty8, lo, hi, chunks):
            a_ts = []
            for (c0, nc, p0, kw) in chunks:
                cs = (jax.lax.broadcasted_iota(jnp.int32, (nc, kw), 0)
                      + (c0 - p0)).astype(jnp.float32)
                dtx = tx[:, p0:p0 + kw] - jnp.float32(p0)
                a_ts.append(jnp.maximum(1.0 - jnp.abs(cs - dtx), 0.0))

            def body(r0):
                rfs = r0.astype(jnp.float32) + ksub.astype(jnp.float32)
                wblk = jnp.maximum(1.0 - jnp.abs(ty8 - rfs), 0.0)
                wblk = jnp.where(r0 + ksub <= hi, wblk, 0.0)
                xparts = []
                for k in range(_DPACK):
                    rf = (r0 + k).astype(jnp.float32)
                    w8 = jnp.maximum(1.0 - jnp.abs(ty8 - rf), 0.0)
                    w8 = jnp.where(r0 + k <= hi, w8, 0.0)
                    wc = jnp.concatenate([w8] * (C // 8), axis=0)
                    xparts.append(xrow * wc)
                v = jnp.concatenate(xparts + [wblk], axis=0)
                idxs = [jnp.where(r0 + k > H - 1, r0 + k - _DPACK, r0 + k)
                        for k in range(_DPACK)]
                xolds = [accx_ref[idxs[k], :, :] for k in range(_DPACK)]
                molds = [accm_ref[idxs[k], 0, :] for k in range(_DPACK)]
                parts = []
                for (c0, nc, p0, kw), a_tc in zip(chunks, a_ts):
                    parts.append(jax.lax.dot_general(
                        v[:, p0:p0 + kw], a_tc,
                        dimension_numbers=(((1,), (1,)), ((), ())),
                        preferred_element_type=jnp.float32))
                r_mat = jnp.concatenate(parts, axis=1)
                for k in range(_DPACK):
                    accx_ref[idxs[k], :, :] = xolds[k] + r_mat[k * C:(k + 1) * C, :]
                for k in range(_DPACK):
                    accm_ref[idxs[k], 0, :] = molds[k] + r_mat[_DPACK * C + k, :]
                return r0 + _DPACK

            jax.lax.while_loop(lambda r0: r0 <= hi, body, lo)

        banded_chunks = []
        for m in range(W // 256):
            c0 = 256 * m
            p0 = max(0, c0 - 128)
            p1 = min(W, c0 + 256 + 128)
            banded_chunks.append((c0, 256, p0, p1 - p0))
        full_chunks = [(0, W, 0, W)]

        for j in range(RB):
            y = i * RB + j
            fx = flow_ref[0, 0, j, :].reshape(1, W)
            fy = flow_ref[0, 1, j, :].reshape(1, W)
            tx = px + fx
            ty = y.astype(jnp.float32) + fy
            ty8 = jnp.broadcast_to(ty, (8, W))
            r1 = jnp.floor(ty)

            xrow = x_ref[0, :, j, :]

            lo = jnp.clip(jnp.min(r1), 0.0, float(H)).astype(jnp.int32)
            hi = jnp.clip(jnp.max(r1) + 1.0, -1.0, float(H - 1)).astype(jnp.int32)

            banded = jnp.max(jnp.abs(fx)) < 127.0

            @pl.when(banded)
            def _banded():
                process_row(xrow, tx, ty8, lo, hi, banded_chunks)

            @pl.when(jnp.logical_not(banded))
            def _full():
                process_row(xrow, tx, ty8, lo, hi, full_chunks)


def kernel(x, upflow):
    B, C, H, W = x.shape
    RB = _ROWS_PER_STEP
    RW = _ROWS_PER_WRITE
    n_acc = H // RB
    n_wr = H // RW
    grid = (B, n_acc + n_wr)
    accx, accm = pl.pallas_call(
        _warp_kernel,
        grid=grid,
        in_specs=[
            pl.BlockSpec((1, C, RB, W),
                         lambda b, i: (b, 0, jnp.minimum(i, n_acc - 1), 0)),
            pl.BlockSpec((1, 2, RB, W),
                         lambda b, i: (b, 0, jnp.minimum(i, n_acc - 1), 0)),
        ],
        out_specs=[
            pl.BlockSpec((1, RW, C, W),
                         lambda b, i: (b, jnp.maximum(i - n_acc, 0), 0, 0)),
            pl.BlockSpec((1, RW, 1, W),
                         lambda b, i: (b, jnp.maximum(i - n_acc, 0), 0, 0)),
        ],
        out_shape=[
            jax.ShapeDtypeStruct((B, H, C, W), jnp.float32),
            jax.ShapeDtypeStruct((B, H, 1, W), jnp.float32),
        ],
        scratch_shapes=[
            pltpu.VMEM((H, C, W), jnp.float32),
            pltpu.VMEM((H, 1, W), jnp.float32),
        ],
        compiler_params=pltpu.CompilerParams(
            dimension_semantics=("arbitrary", "arbitrary")),
    )(x, upflow)
    x_warped = jnp.transpose(accx, (0, 2, 1, 3))
    mask = jnp.transpose(accm, (0, 2, 1, 3))
    return (x_warped, mask, upflow)


# speedup vs baseline: 4.5604x; 1.1786x over previous
import jax
import jax.numpy as jnp
from jax.experimental import pallas as pl
from jax.experimental.pallas import tpu as pltpu

_ROWS_PER_STEP = 8
_ROWS_PER_WRITE = 4
_DPACK = 4
_GROUP = 2


def _warp_kernel(x_ref, flow_ref, outx_ref, outm_ref, accx_ref, accm_ref):
    _, C, RB, W = x_ref.shape
    H = accx_ref.shape[0]
    RW = outx_ref.shape[1]
    n_acc = H // RB
    i = pl.program_id(1)

    @pl.when(i == 0)
    def _init():
        accx_ref[...] = jnp.zeros_like(accx_ref)
        accm_ref[...] = jnp.zeros_like(accm_ref)

    @pl.when(i >= n_acc)
    def _writeout():
        wb = i - n_acc
        outx_ref[0] = accx_ref[pl.ds(wb * RW, RW)]
        outm_ref[0] = accm_ref[pl.ds(wb * RW, RW)]

    @pl.when(i < n_acc)
    def _accumulate():
        px = jax.lax.broadcasted_iota(jnp.int32, (1, W), 1).astype(jnp.float32)
        ksub = jax.lax.broadcasted_iota(jnp.int32, (8, W), 0)

        def process_group(rows, lo, hi, chunks):
            def body(r0):
                rfs = r0.astype(jnp.float32) + ksub.astype(jnp.float32)
                idxs = [jnp.where(r0 + k > H - 1, r0 + k - _DPACK, r0 + k)
                        for k in range(_DPACK)]
                xolds = [accx_ref[idxs[k], :, :] for k in range(_DPACK)]
                molds = [accm_ref[idxs[k], 0, :] for k in range(_DPACK)]
                r_tot = None
                for (xrow, ty8, a_ts) in rows:
                    wblk = jnp.maximum(1.0 - jnp.abs(ty8 - rfs), 0.0)
                    wblk = jnp.where(r0 + ksub <= hi, wblk, 0.0)
                    xparts = []
                    for k in range(_DPACK):
                        rf = (r0 + k).astype(jnp.float32)
                        w8 = jnp.maximum(1.0 - jnp.abs(ty8 - rf), 0.0)
                        w8 = jnp.where(r0 + k <= hi, w8, 0.0)
                        wc = jnp.concatenate([w8] * (C // 8), axis=0)
                        xparts.append(xrow * wc)
                    v = jnp.concatenate(xparts + [wblk], axis=0)
                    parts = []
                    for (c0, nc, p0, kw), a_tc in zip(chunks, a_ts):
                        parts.append(jax.lax.dot_general(
                            v[:, p0:p0 + kw], a_tc,
                            dimension_numbers=(((1,), (1,)), ((), ())),
                            preferred_element_type=jnp.float32))
                    r_mat = jnp.concatenate(parts, axis=1)
                    r_tot = r_mat if r_tot is None else r_tot + r_mat
                for k in range(_DPACK):
                    accx_ref[idxs[k], :, :] = xolds[k] + r_tot[k * C:(k + 1) * C, :]
                for k in range(_DPACK):
                    accm_ref[idxs[k], 0, :] = molds[k] + r_tot[_DPACK * C + k, :]
                return r0 + _DPACK

            jax.lax.while_loop(lambda r0: r0 <= hi, body, lo)

        banded_chunks = []
        for m in range(W // 256):
            c0 = 256 * m
            p0 = max(0, c0 - 128)
            p1 = min(W, c0 + 256 + 128)
            banded_chunks.append((c0, 256, p0, p1 - p0))
        full_chunks = [(0, W, 0, W)]

        banded = jnp.max(jnp.abs(flow_ref[0, 0, :, :])) < 127.0

        def do_rows(chunks):
            for g in range(RB // _GROUP):
                rows = []
                los, his = [], []
                for jj in range(_GROUP):
                    j = g * _GROUP + jj
                    y = i * RB + j
                    fx = flow_ref[0, 0, j, :].reshape(1, W)
                    fy = flow_ref[0, 1, j, :].reshape(1, W)
                    tx = px + fx
                    ty = y.astype(jnp.float32) + fy
                    ty8 = jnp.broadcast_to(ty, (8, W))
                    r1 = jnp.floor(ty)
                    los.append(jnp.min(r1))
                    his.append(jnp.max(r1))
                    xrow = x_ref[0, :, j, :]
                    a_ts = []
                    for (c0, nc, p0, kw) in chunks:
                        cs = (jax.lax.broadcasted_iota(jnp.int32, (nc, kw), 0)
                              + (c0 - p0)).astype(jnp.float32)
                        dtx = tx[:, p0:p0 + kw] - jnp.float32(p0)
                        a_ts.append(jnp.maximum(1.0 - jnp.abs(cs - dtx), 0.0))
                    rows.append((xrow, ty8, a_ts))
                lo = jnp.clip(jnp.min(jnp.stack(los)), 0.0,
                              float(H)).astype(jnp.int32)
                hi = jnp.clip(jnp.max(jnp.stack(his)) + 1.0, -1.0,
                              float(H - 1)).astype(jnp.int32)
                process_group(rows, lo, hi, chunks)

        @pl.when(banded)
        def _banded():
            do_rows(banded_chunks)

        @pl.when(jnp.logical_not(banded))
        def _full():
            do_rows(full_chunks)


def kernel(x, upflow):
    B, C, H, W = x.shape
    RB = _ROWS_PER_STEP
    RW = _ROWS_PER_WRITE
    n_acc = H // RB
    n_wr = H // RW
    grid = (B, n_acc + n_wr)
    accx, accm = pl.pallas_call(
        _warp_kernel,
        grid=grid,
        in_specs=[
            pl.BlockSpec((1, C, RB, W),
                         lambda b, i: (b, 0, jnp.minimum(i, n_acc - 1), 0)),
            pl.BlockSpec((1, 2, RB, W),
                         lambda b, i: (b, 0, jnp.minimum(i, n_acc - 1), 0)),
        ],
        out_specs=[
            pl.BlockSpec((1, RW, C, W),
                         lambda b, i: (b, jnp.maximum(i - n_acc, 0), 0, 0)),
            pl.BlockSpec((1, RW, 1, W),
                         lambda b, i: (b, jnp.maximum(i - n_acc, 0), 0, 0)),
        ],
        out_shape=[
            jax.ShapeDtypeStruct((B, H, C, W), jnp.float32),
            jax.ShapeDtypeStruct((B, H, 1, W), jnp.float32),
        ],
        scratch_shapes=[
            pltpu.VMEM((H, C, W), jnp.float32),
            pltpu.VMEM((H, 1, W), jnp.float32),
        ],
        compiler_params=pltpu.CompilerParams(
            dimension_semantics=("arbitrary", "arbitrary")),
    )(x, upflow)
    x_warped = jnp.transpose(accx, (0, 2, 1, 3))
    mask = jnp.transpose(accm, (0, 2, 1, 3))
    return (x_warped, mask, upflow)
